# baseline (speedup 1.0000x reference)
import functools

import jax
import jax.numpy as jnp
from jax import lax
from jax.experimental import pallas as pl
from jax.experimental.pallas import tpu as pltpu
from jax.experimental.pallas import tpu_sc as plsc

NC = 2
NS = 16
NW = NC * NS
L = 16
K = 80
SCH = 25

_GDN = lax.GatherDimensionNumbers(
    offset_dims=(), collapsed_slice_dims=(0,), start_index_map=(0,))
_PIB = lax.GatherScatterMode.PROMISE_IN_BOUNDS


def _sc_body(nchunks, n, d, row_hbm, col_hbm, a_hbm, x_hbm, part_hbm,
             row_v, col_v, a_v, dum_v, buf0, buf1, buf2,
             semg0, semg1, semg2, sems0, sems1, sems2, acc):
    bufs = (buf0, buf1, buf2)
    semg = (semg0, semg1, semg2)
    sems = (sems0, sems1, sems2)
    cax = lax.axis_index("c")
    s = lax.axis_index("s")
    wid = cax * NS + s
    zch = 80
    base = s * 640
    nzch = jnp.where(s < NS - 1, 640 // zch, (n - 640 * (NS - 1)) // zch)

    zeros = jnp.zeros((L,), jnp.float32)

    def _zero_row(r, _):
        for j in range(d // L):
            buf0[r, pl.ds(j * L, L)] = zeros
        return _

    lax.fori_loop(0, K, _zero_row, 0)

    def _zero_chunk(t, _):
        pltpu.sync_copy(buf0.at[pl.ds(0, zch), :],
                        acc.at[pl.ds(base + t * zch, zch), :])
        return _

    lax.fori_loop(0, nzch, _zero_chunk, 0)
    plsc.subcore_barrier()

    def _scale(cst, b):
        bb = bufs[b]
        for q in range(K // L):
            av16 = a_v[pl.ds(cst * K + q * L, L)]
            for i in range(L):
                av = lax.gather(av16, jnp.full((L, 1), i, jnp.int32),
                                _GDN, (1,), mode=_PIB)
                r = q * L + i
                for j in range(d // L):
                    sl = pl.ds(j * L, L)
                    bb[r, sl] = bb[r, sl] * av

    def _wait_gather(c, b):
        pltpu.make_async_copy(x_hbm.at[col_v.at[c]], bufs[b], semg[b]).wait()

    def _wait_scatter(c, b):
        pltpu.make_async_copy(bufs[b], acc.at[row_v.at[c]], sems[b]).wait()

    def _gather(c, b):
        pltpu.async_copy(x_hbm.at[col_v.at[c]], bufs[b], semg[b])

    def _scatter(c, b):
        pltpu.async_copy(bufs[b], acc.at[row_v.at[c]], sems[b], add=True)

    def _sch(t, _):
        pltpu.sync_copy(row_hbm.at[wid, t], row_v)
        pltpu.sync_copy(col_hbm.at[wid, t], col_v)
        pltpu.sync_copy(a_hbm.at[wid, t], a_v)
        pltpu.async_copy(x_hbm.at[col_v.at[0]], bufs[0], semg[0])
        for c in range(3):
            bn = (c + 1) % 3
            if c == 2:
                _wait_scatter(c - 2, bn)
            _gather(c + 1, bn)
            _wait_gather(c, c)
            _scale(c, c)
            _scatter(c, c)

        def _triple(tr, __):
            c0 = tr * 3
            for o in range(3):
                c = c0 + o
                bn = (o + 1) % 3
                _wait_scatter(c - 2, bn)
                _gather(c + 1, bn)
                _wait_gather(c, o)
                _scale(c, o)
                _scatter(c, o)
            return __

        lax.fori_loop(1, (SCH - 1) // 3, _triple, 0)

        c = SCH - 1
        _wait_gather(c, c % 3)
        _scale(c, c % 3)
        _scatter(c, c % 3)
        for cc in range(SCH - 3, SCH):
            _wait_scatter(cc, cc % 3)
        return _

    lax.fori_loop(0, nchunks // SCH, _sch, 0)

    lax.fori_loop(0, K, _zero_row, 0)

    def _dummy_idx(q, _):
        dum_v[0, pl.ds(q * L, L)] = jnp.full((L,), n + s, jnp.int32)
        return _

    lax.fori_loop(0, K // L, _dummy_idx, 0)
    pltpu.sync_copy(buf0, acc.at[dum_v.at[0]], add=True)
    plsc.subcore_barrier()

    def _spin(i, _):
        buf0[0, pl.ds(0, L)] = buf0[0, pl.ds(0, L)] + zeros
        return _

    lax.fori_loop(0, 5000, _spin, 0)
    plsc.subcore_barrier()

    def _wb_chunk(t, _):
        off = base + t * zch
        pltpu.sync_copy(acc.at[pl.ds(off, zch), :],
                        part_hbm.at[cax, pl.ds(off, zch), :])
        return _

    lax.fori_loop(0, nzch, _wb_chunk, 0)


def _combine_body(p_ref, o_ref):
    o_ref[...] = p_ref[0] + p_ref[1]


def kernel(edge_index, A_values, x):
    n, d = x.shape
    e = A_values.shape[0]
    epw = e // NW
    nchunks = epw // K
    nsch = nchunks // SCH
    assert epw * NW == e and nchunks * K == epw and nsch * SCH == nchunks
    assert 640 * (NS - 1) < n and (n - 640 * (NS - 1)) % 80 == 0
    assert d % L == 0

    row = edge_index[0].astype(jnp.int32).reshape(NW, nsch, SCH, K)
    col = edge_index[1].astype(jnp.int32).reshape(NW, nsch, SCH, K)
    a3 = A_values.reshape(NW, nsch, SCH * K)

    mesh = plsc.VectorSubcoreMesh(core_axis_name="c", subcore_axis_name="s",
                                  num_cores=NC, num_subcores=NS)
    part = pl.kernel(
        functools.partial(_sc_body, nchunks, n, d),
        out_type=jax.ShapeDtypeStruct((NC, n, d), jnp.float32),
        mesh=mesh,
        scratch_types=[
            pltpu.VMEM((SCH, K), jnp.int32),
            pltpu.VMEM((SCH, K), jnp.int32),
            pltpu.VMEM((SCH * K,), jnp.float32),
            pltpu.VMEM((1, K), jnp.int32),
            pltpu.VMEM((K, d), jnp.float32),
            pltpu.VMEM((K, d), jnp.float32),
            pltpu.VMEM((K, d), jnp.float32),
            pltpu.SemaphoreType.DMA,
            pltpu.SemaphoreType.DMA,
            pltpu.SemaphoreType.DMA,
            pltpu.SemaphoreType.DMA,
            pltpu.SemaphoreType.DMA,
            pltpu.SemaphoreType.DMA,
            pltpu.VMEM_SHARED((n + NS, d), jnp.float32),
        ],
    )(row, col, a3, x)

    blk = 2000
    out = pl.pallas_call(
        _combine_body,
        grid=(n // blk,),
        in_specs=[pl.BlockSpec((NC, blk, d), lambda i: (0, i, 0))],
        out_specs=pl.BlockSpec((blk, d), lambda i: (i, 0)),
        out_shape=jax.ShapeDtypeStruct((n, d), jnp.float32),
    )(part)
    return out


# speedup vs baseline: 1.3824x; 1.3824x over previous
import functools

import jax
import jax.numpy as jnp
from jax import lax
from jax.experimental import pallas as pl
from jax.experimental.pallas import tpu as pltpu
from jax.experimental.pallas import tpu_sc as plsc

NC = 2
NS = 16
NW = NC * NS
L = 16
K = 80
SCH = 25

_GDN = lax.GatherDimensionNumbers(
    offset_dims=(), collapsed_slice_dims=(0,), start_index_map=(0,))
_PIB = lax.GatherScatterMode.PROMISE_IN_BOUNDS


def _sc_body(nchunks, n, d, row_hbm, col_hbm, a_hbm, x_hbm, part_hbm,
             row_v, col_v, a_v, dum_v, buf0, buf1, buf2,
             semg0, semg1, semg2, sems0, sems1, sems2, acc):
    bufs = (buf0, buf1, buf2)
    semg = (semg0, semg1, semg2)
    sems = (sems0, sems1, sems2)
    cax = lax.axis_index("c")
    s = lax.axis_index("s")
    wid = cax * NS + s
    zch = 80
    base = s * 640
    nzch = jnp.where(s < NS - 1, 640 // zch, (n - 640 * (NS - 1)) // zch)

    zeros = jnp.zeros((L,), jnp.float32)

    def _zero_row(r, _):
        for j in range(d // L):
            buf0[r, pl.ds(j * L, L)] = zeros
        return _

    lax.fori_loop(0, K, _zero_row, 0)

    def _zero_chunk(t, _):
        pltpu.sync_copy(buf0.at[pl.ds(0, zch), :],
                        acc.at[pl.ds(base + t * zch, zch), :])
        return _

    lax.fori_loop(0, nzch, _zero_chunk, 0)
    plsc.subcore_barrier()

    def _scale(cst, b):
        bb = bufs[b]

        def _group(q, _):
            av16 = a_v[pl.ds(cst * K + q * L, L)]
            for i in range(L):
                av = lax.gather(av16, jnp.full((L, 1), i, jnp.int32),
                                _GDN, (1,), mode=_PIB)
                r = q * L + i
                for j in range(d // L):
                    sl = pl.ds(j * L, L)
                    bb[r, sl] = bb[r, sl] * av
            return _

        lax.fori_loop(0, K // L, _group, 0)

    def _wait_gather(c, b):
        pltpu.make_async_copy(x_hbm.at[col_v.at[c]], bufs[b], semg[b]).wait()

    def _wait_scatter(c, b):
        pltpu.make_async_copy(bufs[b], acc.at[row_v.at[c]], sems[b]).wait()

    def _gather(c, b):
        pltpu.async_copy(x_hbm.at[col_v.at[c]], bufs[b], semg[b])

    def _scatter(c, b):
        pltpu.async_copy(bufs[b], acc.at[row_v.at[c]], sems[b], add=True)

    def _sch(t, _):
        pltpu.sync_copy(row_hbm.at[wid, t], row_v)
        pltpu.sync_copy(col_hbm.at[wid, t], col_v)
        pltpu.sync_copy(a_hbm.at[wid, t], a_v)
        pltpu.async_copy(x_hbm.at[col_v.at[0]], bufs[0], semg[0])
        for c in range(3):
            bn = (c + 1) % 3
            if c == 2:
                _wait_scatter(c - 2, bn)
            _gather(c + 1, bn)
            _wait_gather(c, c)
            _scale(c, c)
            _scatter(c, c)

        def _triple(tr, __):
            c0 = tr * 3
            for o in range(3):
                c = c0 + o
                bn = (o + 1) % 3
                _wait_scatter(c - 2, bn)
                _gather(c + 1, bn)
                _wait_gather(c, o)
                _scale(c, o)
                _scatter(c, o)
            return __

        lax.fori_loop(1, (SCH - 1) // 3, _triple, 0)

        c = SCH - 1
        _wait_gather(c, c % 3)
        _scale(c, c % 3)
        _scatter(c, c % 3)
        for cc in range(SCH - 3, SCH):
            _wait_scatter(cc, cc % 3)
        return _

    lax.fori_loop(0, nchunks // SCH, _sch, 0)

    lax.fori_loop(0, K, _zero_row, 0)

    def _dummy_idx(q, _):
        dum_v[0, pl.ds(q * L, L)] = jnp.full((L,), n + s, jnp.int32)
        return _

    lax.fori_loop(0, K // L, _dummy_idx, 0)
    pltpu.sync_copy(buf0, acc.at[dum_v.at[0]], add=True)
    plsc.subcore_barrier()

    def _spin(i, _):
        buf0[0, pl.ds(0, L)] = buf0[0, pl.ds(0, L)] + zeros
        return _

    lax.fori_loop(0, 5000, _spin, 0)
    plsc.subcore_barrier()

    def _wb_chunk(t, _):
        off = base + t * zch
        pltpu.sync_copy(acc.at[pl.ds(off, zch), :],
                        part_hbm.at[cax, pl.ds(off, zch), :])
        return _

    lax.fori_loop(0, nzch, _wb_chunk, 0)


def _combine_body(p_ref, o_ref):
    o_ref[...] = p_ref[0] + p_ref[1]


def kernel(edge_index, A_values, x):
    n, d = x.shape
    e = A_values.shape[0]
    epw = e // NW
    nchunks = epw // K
    nsch = nchunks // SCH
    assert epw * NW == e and nchunks * K == epw and nsch * SCH == nchunks
    assert 640 * (NS - 1) < n and (n - 640 * (NS - 1)) % 80 == 0
    assert d % L == 0

    row = edge_index[0].astype(jnp.int32).reshape(NW, nsch, SCH, K)
    col = edge_index[1].astype(jnp.int32).reshape(NW, nsch, SCH, K)
    a3 = A_values.reshape(NW, nsch, SCH * K)

    mesh = plsc.VectorSubcoreMesh(core_axis_name="c", subcore_axis_name="s",
                                  num_cores=NC, num_subcores=NS)
    part = pl.kernel(
        functools.partial(_sc_body, nchunks, n, d),
        out_type=jax.ShapeDtypeStruct((NC, n, d), jnp.float32),
        mesh=mesh,
        scratch_types=[
            pltpu.VMEM((SCH, K), jnp.int32),
            pltpu.VMEM((SCH, K), jnp.int32),
            pltpu.VMEM((SCH * K,), jnp.float32),
            pltpu.VMEM((1, K), jnp.int32),
            pltpu.VMEM((K, d), jnp.float32),
            pltpu.VMEM((K, d), jnp.float32),
            pltpu.VMEM((K, d), jnp.float32),
            pltpu.SemaphoreType.DMA,
            pltpu.SemaphoreType.DMA,
            pltpu.SemaphoreType.DMA,
            pltpu.SemaphoreType.DMA,
            pltpu.SemaphoreType.DMA,
            pltpu.SemaphoreType.DMA,
            pltpu.VMEM_SHARED((n + NS, d), jnp.float32),
        ],
    )(row, col, a3, x)

    blk = 2000
    out = pl.pallas_call(
        _combine_body,
        grid=(n // blk,),
        in_specs=[pl.BlockSpec((NC, blk, d), lambda i: (0, i, 0))],
        out_specs=pl.BlockSpec((blk, d), lambda i: (i, 0)),
        out_shape=jax.ShapeDtypeStruct((n, d), jnp.float32),
    )(part)
    return out


# speedup vs baseline: 1.4344x; 1.0375x over previous
import functools

import jax
import jax.numpy as jnp
from jax import lax
from jax.experimental import pallas as pl
from jax.experimental.pallas import tpu as pltpu
from jax.experimental.pallas import tpu_sc as plsc

NC = 2
NS = 16
NW = NC * NS
L = 16
K = 80
SCH = 25

_GDN = lax.GatherDimensionNumbers(
    offset_dims=(), collapsed_slice_dims=(0,), start_index_map=(0,))
_PIB = lax.GatherScatterMode.PROMISE_IN_BOUNDS


def _sc_body(nchunks, n, d, row_hbm, col_hbm, a_hbm, x_hbm, part_hbm,
             row_v, col_v, a_v, dum_v, buf0, buf1, buf2,
             semg0, semg1, semg2, sems0, sems1, sems2, acc):
    bufs = (buf0, buf1, buf2)
    semg = (semg0, semg1, semg2)
    sems = (sems0, sems1, sems2)
    cax = lax.axis_index("c")
    s = lax.axis_index("s")
    wid = cax * NS + s
    zch = 80
    base = s * 640
    nzch = jnp.where(s < NS - 1, 640 // zch, (n - 640 * (NS - 1)) // zch)

    zeros = jnp.zeros((L,), jnp.float32)

    def _zero_row(r, _):
        for j in range(d // L):
            buf0[r, pl.ds(j * L, L)] = zeros
        return _

    lax.fori_loop(0, K, _zero_row, 0)

    def _zero_chunk(t, _):
        pltpu.async_copy(buf0.at[pl.ds(0, zch), :],
                         acc.at[pl.ds(base + t * zch, zch), :], semg0)
        return _

    lax.fori_loop(0, nzch, _zero_chunk, 0)

    def _zero_wait(t, _):
        pltpu.make_async_copy(buf0.at[pl.ds(0, zch), :],
                              acc.at[pl.ds(base + t * zch, zch), :],
                              semg0).wait()
        return _

    lax.fori_loop(0, nzch, _zero_wait, 0)
    plsc.subcore_barrier()

    def _scale(cst, b):
        bb = bufs[b]

        def _group(q, _):
            av16 = a_v[pl.ds(cst * K + q * L, L)]
            for i in range(L):
                av = lax.gather(av16, jnp.full((L, 1), i, jnp.int32),
                                _GDN, (1,), mode=_PIB)
                r = q * L + i
                for j in range(d // L):
                    sl = pl.ds(j * L, L)
                    bb[r, sl] = bb[r, sl] * av
            return _

        lax.fori_loop(0, K // L, _group, 0)

    def _wait_gather(c, b):
        pltpu.make_async_copy(x_hbm.at[col_v.at[c]], bufs[b], semg[b]).wait()

    def _wait_scatter(c, b):
        pltpu.make_async_copy(bufs[b], acc.at[row_v.at[c]], sems[b]).wait()

    def _gather(c, b):
        pltpu.async_copy(x_hbm.at[col_v.at[c]], bufs[b], semg[b])

    def _scatter(c, b):
        pltpu.async_copy(bufs[b], acc.at[row_v.at[c]], sems[b], add=True)

    def _sch(t, _):
        pltpu.async_copy(row_hbm.at[wid, t], row_v, sems[0])
        pltpu.async_copy(col_hbm.at[wid, t], col_v, sems[1])
        pltpu.async_copy(a_hbm.at[wid, t], a_v, sems[2])
        pltpu.make_async_copy(row_hbm.at[wid, t], row_v, sems[0]).wait()
        pltpu.make_async_copy(col_hbm.at[wid, t], col_v, sems[1]).wait()
        pltpu.make_async_copy(a_hbm.at[wid, t], a_v, sems[2]).wait()
        pltpu.async_copy(x_hbm.at[col_v.at[0]], bufs[0], semg[0])
        for c in range(3):
            bn = (c + 1) % 3
            if c == 2:
                _wait_scatter(c - 2, bn)
            _gather(c + 1, bn)
            _wait_gather(c, c)
            _scale(c, c)
            _scatter(c, c)

        def _triple(tr, __):
            c0 = tr * 3
            for o in range(3):
                c = c0 + o
                bn = (o + 1) % 3
                _wait_scatter(c - 2, bn)
                _gather(c + 1, bn)
                _wait_gather(c, o)
                _scale(c, o)
                _scatter(c, o)
            return __

        lax.fori_loop(1, (SCH - 1) // 3, _triple, 0)

        c = SCH - 1
        _wait_gather(c, c % 3)
        _scale(c, c % 3)
        _scatter(c, c % 3)
        for cc in range(SCH - 3, SCH):
            _wait_scatter(cc, cc % 3)
        return _

    lax.fori_loop(0, nchunks // SCH, _sch, 0)

    lax.fori_loop(0, K, _zero_row, 0)

    def _dummy_idx(q, _):
        dum_v[0, pl.ds(q * L, L)] = jnp.full((L,), n + s, jnp.int32)
        return _

    lax.fori_loop(0, K // L, _dummy_idx, 0)
    pltpu.sync_copy(buf0, acc.at[dum_v.at[0]], add=True)
    plsc.subcore_barrier()

    def _spin(i, _):
        buf0[0, pl.ds(0, L)] = buf0[0, pl.ds(0, L)] + zeros
        return _

    lax.fori_loop(0, 5000, _spin, 0)
    plsc.subcore_barrier()

    def _wb_chunk(t, _):
        off = base + t * zch
        pltpu.sync_copy(acc.at[pl.ds(off, zch), :],
                        part_hbm.at[cax, pl.ds(off, zch), :])
        return _

    lax.fori_loop(0, nzch, _wb_chunk, 0)


def _combine_body(p_ref, o_ref):
    o_ref[...] = p_ref[0] + p_ref[1]


def kernel(edge_index, A_values, x):
    n, d = x.shape
    e = A_values.shape[0]
    epw = e // NW
    nchunks = epw // K
    nsch = nchunks // SCH
    assert epw * NW == e and nchunks * K == epw and nsch * SCH == nchunks
    assert 640 * (NS - 1) < n and (n - 640 * (NS - 1)) % 80 == 0
    assert d % L == 0

    row = edge_index[0].astype(jnp.int32).reshape(NW, nsch, SCH, K)
    col = edge_index[1].astype(jnp.int32).reshape(NW, nsch, SCH, K)
    a3 = A_values.reshape(NW, nsch, SCH * K)

    mesh = plsc.VectorSubcoreMesh(core_axis_name="c", subcore_axis_name="s",
                                  num_cores=NC, num_subcores=NS)
    part = pl.kernel(
        functools.partial(_sc_body, nchunks, n, d),
        out_type=jax.ShapeDtypeStruct((NC, n, d), jnp.float32),
        mesh=mesh,
        scratch_types=[
            pltpu.VMEM((SCH, K), jnp.int32),
            pltpu.VMEM((SCH, K), jnp.int32),
            pltpu.VMEM((SCH * K,), jnp.float32),
            pltpu.VMEM((1, K), jnp.int32),
            pltpu.VMEM((K, d), jnp.float32),
            pltpu.VMEM((K, d), jnp.float32),
            pltpu.VMEM((K, d), jnp.float32),
            pltpu.SemaphoreType.DMA,
            pltpu.SemaphoreType.DMA,
            pltpu.SemaphoreType.DMA,
            pltpu.SemaphoreType.DMA,
            pltpu.SemaphoreType.DMA,
            pltpu.SemaphoreType.DMA,
            pltpu.VMEM_SHARED((n + NS, d), jnp.float32),
        ],
    )(row, col, a3, x)

    blk = 2000
    out = pl.pallas_call(
        _combine_body,
        grid=(n // blk,),
        in_specs=[pl.BlockSpec((NC, blk, d), lambda i: (0, i, 0))],
        out_specs=pl.BlockSpec((blk, d), lambda i: (i, 0)),
        out_shape=jax.ShapeDtypeStruct((n, d), jnp.float32),
    )(part)
    return out
